# baseline (speedup 1.0000x reference)
import functools

import jax
import jax.numpy as jnp
from jax import lax
from jax.experimental import pallas as pl
from jax.experimental.pallas import tpu as pltpu
from jax.experimental.pallas import tpu_sc as plsc

G = 64
NC = 2
NS = 16
CHUNK = 128
WDEG = 128


def _sc_mesh():
    return plsc.VectorSubcoreMesh(
        core_axis_name="c", subcore_axis_name="s", num_cores=NC, num_subcores=NS
    )


def _make_deg(n_pad, nck, rpt):
    @functools.partial(
        pl.kernel,
        out_type=jax.ShapeDtypeStruct((NC, n_pad, WDEG), jnp.float32),
        mesh=_sc_mesh(),
        scratch_types=[
            pltpu.VMEM((CHUNK,), jnp.int32),
            pltpu.VMEM((CHUNK, WDEG), jnp.float32),
            pltpu.VMEM_SHARED((n_pad, WDEG), jnp.float32),
        ],
    )
    def deg_kernel(dst_hbm, ones_hbm, zdeg_hbm, out_hbm, didx, ones_v, acc):
        c = lax.axis_index("c")
        s = lax.axis_index("s")
        pltpu.sync_copy(zdeg_hbm, acc.at[pl.ds(s * rpt, rpt)])
        pltpu.sync_copy(ones_hbm, ones_v)
        plsc.subcore_barrier()
        nchunk = lax.select(c == 0, nck[0], nck[1])
        core0_edges = NS * nck[0] * CHUNK
        base = lax.select(c == 0, s * nck[0] * CHUNK,
                          core0_edges + s * nck[1] * CHUNK)

        @pl.loop(0, nchunk)
        def _(i):
            off = pl.multiple_of(base + i * CHUNK, CHUNK)
            pltpu.sync_copy(dst_hbm.at[pl.ds(off, CHUNK)], didx)
            pltpu.sync_copy(ones_v, acc.at[didx], add=True)

        plsc.subcore_barrier()
        pltpu.sync_copy(acc.at[pl.ds(s * rpt, rpt)],
                        out_hbm.at[c, pl.ds(s * rpt, rpt)])

    return deg_kernel


def _make_scatter(n_pad, d, nck, rpt):
    @functools.partial(
        pl.kernel,
        out_type=jax.ShapeDtypeStruct((NC, n_pad, d), jnp.float32),
        mesh=_sc_mesh(),
        scratch_types=[
            pltpu.VMEM((CHUNK,), jnp.int32),
            pltpu.VMEM((CHUNK,), jnp.int32),
            pltpu.VMEM((CHUNK,), jnp.int32),
            pltpu.VMEM((CHUNK,), jnp.int32),
            pltpu.VMEM((CHUNK, d), jnp.float32),
            pltpu.VMEM((CHUNK, d), jnp.float32),
            pltpu.VMEM_SHARED((n_pad, d), jnp.float32),
            pltpu.SemaphoreType.DMA,
            pltpu.SemaphoreType.DMA,
        ],
    )
    def scat_kernel(src_hbm, dst_hbm, y_hbm, zrow_hbm, out_hbm,
                    sidxa, didxa, sidxb, didxb, rowsa, rowsb, acc,
                    sema, semb):
        c = lax.axis_index("c")
        s = lax.axis_index("s")
        pltpu.sync_copy(zrow_hbm, acc.at[pl.ds(s * rpt, rpt)])
        plsc.subcore_barrier()
        nch2 = lax.select(c == 0, nck[0] // 2, nck[1] // 2)
        core0_edges = NS * nck[0] * CHUNK
        base = lax.select(c == 0, s * nck[0] * CHUNK,
                          core0_edges + s * nck[1] * CHUNK)

        @pl.loop(0, nch2)
        def _(g):
            offa = pl.multiple_of(base + (2 * g) * CHUNK, CHUNK)
            pltpu.sync_copy(src_hbm.at[pl.ds(offa, CHUNK)], sidxa)
            pltpu.sync_copy(dst_hbm.at[pl.ds(offa, CHUNK)], didxa)
            cpa = pltpu.async_copy(y_hbm.at[sidxa], rowsa, sema)
            offb = pl.multiple_of(base + (2 * g + 1) * CHUNK, CHUNK)
            pltpu.sync_copy(src_hbm.at[pl.ds(offb, CHUNK)], sidxb)
            pltpu.sync_copy(dst_hbm.at[pl.ds(offb, CHUNK)], didxb)
            cpb = pltpu.async_copy(y_hbm.at[sidxb], rowsb, semb)
            cpa.wait()
            pltpu.sync_copy(rowsa, acc.at[didxa], add=True)
            cpb.wait()
            pltpu.sync_copy(rowsb, acc.at[didxb], add=True)

        plsc.subcore_barrier()
        pltpu.sync_copy(acc.at[pl.ds(s * rpt, rpt)],
                        out_hbm.at[c, pl.ds(s * rpt, rpt)])

    return scat_kernel


def _dinv_from_parts(degp):
    deg = degp[0, :, 0:1] + degp[1, :, 0:1] + 1.0
    return lax.rsqrt(deg)


def _xw_body(x_ref, w1_ref, xw_ref):
    xw_ref[...] = jnp.dot(x_ref[...], w1_ref[...],
                          preferred_element_type=jnp.float32)


def _scale_body(xw_ref, degp_ref, y_ref):
    y_ref[...] = xw_ref[...] * _dinv_from_parts(degp_ref[...])


def _lin2_body(y1_ref, accp_ref, degp_ref, w2_ref, b1_ref, y2_ref):
    dinv = _dinv_from_parts(degp_ref[...])
    acc = accp_ref[0] + accp_ref[1] + y1_ref[...]
    h1 = jax.nn.relu(dinv * acc + b1_ref[...])
    y2_ref[...] = jnp.dot(h1, w2_ref[...], preferred_element_type=jnp.float32) * dinv


def _make_final(n, n_pad):
    def final_body(y2_ref, accp_ref, degp_ref, b2_ref, batch_ref, wl_ref,
                   bl_ref, out_ref):
        dinv = _dinv_from_parts(degp_ref[...])
        acc = accp_ref[0] + accp_ref[1] + y2_ref[...]
        h2 = jax.nn.relu(dinv * acc + b2_ref[...])[:n]
        seg = lax.broadcasted_iota(jnp.int32, (1, G), 1)
        mask = (batch_ref[...] == seg).astype(jnp.float32)
        dn = (((0,), (0,)), ((), ()))
        sums = lax.dot_general(mask, h2, dn, preferred_element_type=jnp.float32)
        cnt = lax.dot_general(mask, jnp.ones((n, 1), jnp.float32), dn,
                              preferred_element_type=jnp.float32)
        pooled = sums / jnp.maximum(cnt, 1.0)
        out_ref[...] = (
            jnp.dot(pooled, wl_ref[...], preferred_element_type=jnp.float32)
            + bl_ref[...]
        )

    return final_body


def kernel(x, edge_index, batch, W1, b1, W2, b2, Wl, bl):
    n, d = x.shape
    e = edge_index.shape[1]

    n_pad = ((n + (8 * NS) - 1) // (8 * NS)) * (8 * NS)
    rpt = n_pad // NS
    cpp = (e + NS * CHUNK - 1) // (NS * CHUNK)
    n1 = ((cpp * 42) // 100 + 1) // 2 * 2
    n0 = (cpp - n1 + 1) // 2 * 2
    nck_scat = (n0, n1)
    e_pad = NS * CHUNK * max(cpp, n0 + n1) + CHUNK
    nck_deg = ((cpp + 1) // 2, cpp // 2)

    pad_e = e_pad - e
    src = jnp.concatenate(
        [edge_index[0], jnp.full((pad_e,), n, dtype=jnp.int32)])
    dst = jnp.concatenate(
        [edge_index[1], jnp.full((pad_e,), n, dtype=jnp.int32)])
    x_ext = jnp.concatenate(
        [x, jnp.zeros((n_pad - n, d), dtype=jnp.float32)], axis=0)
    ones_deg = jnp.ones((CHUNK, WDEG), dtype=jnp.float32)
    zeros_deg = jnp.zeros((rpt, WDEG), dtype=jnp.float32)
    zeros_row = jnp.zeros((rpt, d), dtype=jnp.float32)
    batch2 = batch.reshape(n, 1)
    b1r = b1.reshape(1, d)
    b2r = b2.reshape(1, d)
    blr = bl.reshape(1, 1)

    degp = _make_deg(n_pad, nck_deg, rpt)(dst, ones_deg, zeros_deg)

    xw1 = pl.pallas_call(
        _xw_body,
        out_shape=jax.ShapeDtypeStruct((n_pad, d), jnp.float32),
    )(x_ext, W1)
    y1 = pl.pallas_call(
        _scale_body,
        out_shape=jax.ShapeDtypeStruct((n_pad, d), jnp.float32),
    )(xw1, degp)

    scat = _make_scatter(n_pad, d, nck_scat, rpt)

    acc1 = scat(src, dst, y1, zeros_row)

    y2 = pl.pallas_call(
        _lin2_body,
        out_shape=jax.ShapeDtypeStruct((n_pad, d), jnp.float32),
    )(y1, acc1, degp, W2, b1r)

    acc2 = scat(src, dst, y2, zeros_row)

    out = pl.pallas_call(
        _make_final(n, n_pad),
        out_shape=jax.ShapeDtypeStruct((G, 1), jnp.float32),
    )(y2, acc2, degp, b2r, batch2, Wl, blr)

    return out.reshape(-1)


# speedup vs baseline: 1.0707x; 1.0707x over previous
import functools

import jax
import jax.numpy as jnp
from jax import lax
from jax.experimental import pallas as pl
from jax.experimental.pallas import tpu as pltpu
from jax.experimental.pallas import tpu_sc as plsc

G = 64
NC = 2
NS = 16
CHUNK = 128
WDEG = 128


def _sc_mesh():
    return plsc.VectorSubcoreMesh(
        core_axis_name="c", subcore_axis_name="s", num_cores=NC, num_subcores=NS
    )


def _make_deg(n_pad, nck, rpt):
    @functools.partial(
        pl.kernel,
        out_type=jax.ShapeDtypeStruct((NC, n_pad, WDEG), jnp.float32),
        mesh=_sc_mesh(),
        scratch_types=[
            pltpu.VMEM((CHUNK,), jnp.int32),
            pltpu.VMEM((CHUNK, WDEG), jnp.float32),
            pltpu.VMEM_SHARED((n_pad, WDEG), jnp.float32),
        ],
    )
    def deg_kernel(dst_hbm, ones_hbm, zdeg_hbm, out_hbm, didx, ones_v, acc):
        c = lax.axis_index("c")
        s = lax.axis_index("s")
        pltpu.sync_copy(zdeg_hbm, acc.at[pl.ds(s * rpt, rpt)])
        pltpu.sync_copy(ones_hbm, ones_v)
        plsc.subcore_barrier()
        nchunk = lax.select(c == 0, nck[0], nck[1])
        core0_edges = NS * nck[0] * CHUNK
        base = lax.select(c == 0, s * nck[0] * CHUNK,
                          core0_edges + s * nck[1] * CHUNK)

        @pl.loop(0, nchunk)
        def _(i):
            off = pl.multiple_of(base + i * CHUNK, CHUNK)
            pltpu.sync_copy(dst_hbm.at[pl.ds(off, CHUNK)], didx)
            pltpu.sync_copy(ones_v, acc.at[didx], add=True)

        plsc.subcore_barrier()
        pltpu.sync_copy(acc.at[pl.ds(s * rpt, rpt)],
                        out_hbm.at[c, pl.ds(s * rpt, rpt)])

    return deg_kernel


def _make_scatter(n_pad, d, nck, rpt):
    @functools.partial(
        pl.kernel,
        out_type=jax.ShapeDtypeStruct((NC, n_pad, d), jnp.float32),
        mesh=_sc_mesh(),
        scratch_types=[
            pltpu.VMEM((CHUNK,), jnp.int32),
            pltpu.VMEM((CHUNK,), jnp.int32),
            pltpu.VMEM((CHUNK,), jnp.int32),
            pltpu.VMEM((CHUNK,), jnp.int32),
            pltpu.VMEM((CHUNK, d), jnp.float32),
            pltpu.VMEM((CHUNK, d), jnp.float32),
            pltpu.VMEM_SHARED((n_pad, d), jnp.float32),
            pltpu.SemaphoreType.DMA,
            pltpu.SemaphoreType.DMA,
        ],
    )
    def scat_kernel(src_hbm, dst_hbm, y_hbm, zrow_hbm, out_hbm,
                    sidxa, didxa, sidxb, didxb, rowsa, rowsb, acc,
                    sema, semb):
        c = lax.axis_index("c")
        s = lax.axis_index("s")
        pltpu.sync_copy(zrow_hbm, acc.at[pl.ds(s * rpt, rpt)])
        plsc.subcore_barrier()
        nch2 = lax.select(c == 0, nck[0] // 2, nck[1] // 2)
        core0_edges = NS * nck[0] * CHUNK
        base = lax.select(c == 0, s * nck[0] * CHUNK,
                          core0_edges + s * nck[1] * CHUNK)

        @pl.loop(0, nch2)
        def _(g):
            offa = pl.multiple_of(base + (2 * g) * CHUNK, CHUNK)
            pltpu.sync_copy(src_hbm.at[pl.ds(offa, CHUNK)], sidxa)
            pltpu.sync_copy(dst_hbm.at[pl.ds(offa, CHUNK)], didxa)
            cpa = pltpu.async_copy(y_hbm.at[sidxa], rowsa, sema)
            offb = pl.multiple_of(base + (2 * g + 1) * CHUNK, CHUNK)
            pltpu.sync_copy(src_hbm.at[pl.ds(offb, CHUNK)], sidxb)
            pltpu.sync_copy(dst_hbm.at[pl.ds(offb, CHUNK)], didxb)
            cpb = pltpu.async_copy(y_hbm.at[sidxb], rowsb, semb)
            cpa.wait()
            pltpu.sync_copy(rowsa, acc.at[didxa], add=True)
            cpb.wait()
            pltpu.sync_copy(rowsb, acc.at[didxb], add=True)

        plsc.subcore_barrier()
        pltpu.sync_copy(acc.at[pl.ds(s * rpt, rpt)],
                        out_hbm.at[c, pl.ds(s * rpt, rpt)])

    return scat_kernel


def _dinv_from_parts(degp):
    deg = degp[0, :, 0:1] + degp[1, :, 0:1] + 1.0
    return lax.rsqrt(deg)


def _xw_body(x_ref, w1_ref, xw_ref):
    xw_ref[...] = jnp.dot(x_ref[...], w1_ref[...],
                          preferred_element_type=jnp.float32)


def _scale_body(xw_ref, degp_ref, y_ref):
    y_ref[...] = xw_ref[...] * _dinv_from_parts(degp_ref[...])


def _lin2_body(y1_ref, accp_ref, degp_ref, w2_ref, b1_ref, y2_ref):
    dinv = _dinv_from_parts(degp_ref[...])
    acc = accp_ref[0] + accp_ref[1] + y1_ref[...]
    h1 = jax.nn.relu(dinv * acc + b1_ref[...])
    y2_ref[...] = jnp.dot(h1, w2_ref[...], preferred_element_type=jnp.float32) * dinv


def _make_final(n, n_pad):
    def final_body(y2_ref, accp_ref, degp_ref, b2_ref, batch_ref, wl_ref,
                   bl_ref, out_ref):
        dinv = _dinv_from_parts(degp_ref[...])
        acc = accp_ref[0] + accp_ref[1] + y2_ref[...]
        h2 = jax.nn.relu(dinv * acc + b2_ref[...])[:n]
        seg = lax.broadcasted_iota(jnp.int32, (1, G), 1)
        mask = (batch_ref[...] == seg).astype(jnp.float32)
        dn = (((0,), (0,)), ((), ()))
        sums = lax.dot_general(mask, h2, dn, preferred_element_type=jnp.float32)
        cnt = lax.dot_general(mask, jnp.ones((n, 1), jnp.float32), dn,
                              preferred_element_type=jnp.float32)
        pooled = sums / jnp.maximum(cnt, 1.0)
        out_ref[...] = (
            jnp.dot(pooled, wl_ref[...], preferred_element_type=jnp.float32)
            + bl_ref[...]
        )

    return final_body


def kernel(x, edge_index, batch, W1, b1, W2, b2, Wl, bl):
    n, d = x.shape
    e = edge_index.shape[1]

    n_pad = ((n + (8 * NS) - 1) // (8 * NS)) * (8 * NS)
    rpt = n_pad // NS
    cpp = (e + NS * CHUNK - 1) // (NS * CHUNK)
    n1 = ((cpp * 32) // 100 + 1) // 2 * 2
    n0 = (cpp - n1 + 1) // 2 * 2
    nck_scat = (n0, n1)
    e_pad = NS * CHUNK * max(cpp, n0 + n1) + CHUNK
    nck_deg = ((cpp + 1) // 2, cpp // 2)

    pad_e = e_pad - e
    src = jnp.concatenate(
        [edge_index[0], jnp.full((pad_e,), n, dtype=jnp.int32)])
    dst = jnp.concatenate(
        [edge_index[1], jnp.full((pad_e,), n, dtype=jnp.int32)])
    x_ext = jnp.concatenate(
        [x, jnp.zeros((n_pad - n, d), dtype=jnp.float32)], axis=0)
    ones_deg = jnp.ones((CHUNK, WDEG), dtype=jnp.float32)
    zeros_deg = jnp.zeros((rpt, WDEG), dtype=jnp.float32)
    zeros_row = jnp.zeros((rpt, d), dtype=jnp.float32)
    batch2 = batch.reshape(n, 1)
    b1r = b1.reshape(1, d)
    b2r = b2.reshape(1, d)
    blr = bl.reshape(1, 1)

    degp = _make_deg(n_pad, nck_deg, rpt)(dst, ones_deg, zeros_deg)

    xw1 = pl.pallas_call(
        _xw_body,
        out_shape=jax.ShapeDtypeStruct((n_pad, d), jnp.float32),
    )(x_ext, W1)
    y1 = pl.pallas_call(
        _scale_body,
        out_shape=jax.ShapeDtypeStruct((n_pad, d), jnp.float32),
    )(xw1, degp)

    scat = _make_scatter(n_pad, d, nck_scat, rpt)

    acc1 = scat(src, dst, y1, zeros_row)

    y2 = pl.pallas_call(
        _lin2_body,
        out_shape=jax.ShapeDtypeStruct((n_pad, d), jnp.float32),
    )(y1, acc1, degp, W2, b1r)

    acc2 = scat(src, dst, y2, zeros_row)

    out = pl.pallas_call(
        _make_final(n, n_pad),
        out_shape=jax.ShapeDtypeStruct((G, 1), jnp.float32),
    )(y2, acc2, degp, b2r, batch2, Wl, blr)

    return out.reshape(-1)


# speedup vs baseline: 1.0881x; 1.0163x over previous
import functools

import jax
import jax.numpy as jnp
from jax import lax
from jax.experimental import pallas as pl
from jax.experimental.pallas import tpu as pltpu
from jax.experimental.pallas import tpu_sc as plsc

G = 64
NC = 2
NS = 16
CHUNK = 128
WDEG = 128


def _sc_mesh():
    return plsc.VectorSubcoreMesh(
        core_axis_name="c", subcore_axis_name="s", num_cores=NC, num_subcores=NS
    )


def _make_deg(n_pad, nck, rpt):
    @functools.partial(
        pl.kernel,
        out_type=jax.ShapeDtypeStruct((NC, n_pad, WDEG), jnp.float32),
        mesh=_sc_mesh(),
        scratch_types=[
            pltpu.VMEM((CHUNK,), jnp.int32),
            pltpu.VMEM((CHUNK, WDEG), jnp.float32),
            pltpu.VMEM_SHARED((n_pad, WDEG), jnp.float32),
        ],
    )
    def deg_kernel(dst_hbm, ones_hbm, zdeg_hbm, out_hbm, didx, ones_v, acc):
        c = lax.axis_index("c")
        s = lax.axis_index("s")
        pltpu.sync_copy(zdeg_hbm, acc.at[pl.ds(s * rpt, rpt)])
        pltpu.sync_copy(ones_hbm, ones_v)
        plsc.subcore_barrier()
        nchunk = lax.select(c == 0, nck[0], nck[1])
        core0_edges = NS * nck[0] * CHUNK
        base = lax.select(c == 0, s * nck[0] * CHUNK,
                          core0_edges + s * nck[1] * CHUNK)

        @pl.loop(0, nchunk)
        def _(i):
            off = pl.multiple_of(base + i * CHUNK, CHUNK)
            pltpu.sync_copy(dst_hbm.at[pl.ds(off, CHUNK)], didx)
            pltpu.sync_copy(ones_v, acc.at[didx], add=True)

        plsc.subcore_barrier()
        pltpu.sync_copy(acc.at[pl.ds(s * rpt, rpt)],
                        out_hbm.at[c, pl.ds(s * rpt, rpt)])

    return deg_kernel


def _make_scatter(n_pad, d, nck, rpt):
    @functools.partial(
        pl.kernel,
        out_type=jax.ShapeDtypeStruct((NC, n_pad, d), jnp.float32),
        mesh=_sc_mesh(),
        scratch_types=[
            pltpu.VMEM((CHUNK,), jnp.int32),
            pltpu.VMEM((CHUNK,), jnp.int32),
            pltpu.VMEM((CHUNK,), jnp.int32),
            pltpu.VMEM((CHUNK,), jnp.int32),
            pltpu.VMEM((CHUNK, d), jnp.float32),
            pltpu.VMEM((CHUNK, d), jnp.float32),
            pltpu.VMEM_SHARED((n_pad, d), jnp.float32),
            pltpu.SemaphoreType.DMA,
            pltpu.SemaphoreType.DMA,
        ],
    )
    def scat_kernel(src_hbm, dst_hbm, y_hbm, zrow_hbm, out_hbm,
                    sidxa, didxa, sidxb, didxb, rowsa, rowsb, acc,
                    sema, semb):
        c = lax.axis_index("c")
        s = lax.axis_index("s")
        pltpu.sync_copy(zrow_hbm, acc.at[pl.ds(s * rpt, rpt)])
        plsc.subcore_barrier()
        nch2 = lax.select(c == 0, nck[0] // 2, nck[1] // 2)
        core0_edges = NS * nck[0] * CHUNK
        base = lax.select(c == 0, s * nck[0] * CHUNK,
                          core0_edges + s * nck[1] * CHUNK)

        @pl.loop(0, nch2)
        def _(g):
            offa = pl.multiple_of(base + (2 * g) * CHUNK, CHUNK)
            pltpu.sync_copy(src_hbm.at[pl.ds(offa, CHUNK)], sidxa)
            pltpu.sync_copy(dst_hbm.at[pl.ds(offa, CHUNK)], didxa)
            cpa = pltpu.async_copy(y_hbm.at[sidxa], rowsa, sema)
            offb = pl.multiple_of(base + (2 * g + 1) * CHUNK, CHUNK)
            pltpu.sync_copy(src_hbm.at[pl.ds(offb, CHUNK)], sidxb)
            pltpu.sync_copy(dst_hbm.at[pl.ds(offb, CHUNK)], didxb)
            cpb = pltpu.async_copy(y_hbm.at[sidxb], rowsb, semb)
            cpa.wait()
            pltpu.sync_copy(rowsa, acc.at[didxa], add=True)
            cpb.wait()
            pltpu.sync_copy(rowsb, acc.at[didxb], add=True)

        plsc.subcore_barrier()
        pltpu.sync_copy(acc.at[pl.ds(s * rpt, rpt)],
                        out_hbm.at[c, pl.ds(s * rpt, rpt)])

    return scat_kernel


def _dinv_from_parts(degp):
    deg = degp[0, :, 0:1] + degp[1, :, 0:1] + 1.0
    return lax.rsqrt(deg)


def _xw_body(x_ref, w1_ref, xw_ref):
    xw_ref[...] = jnp.dot(x_ref[...], w1_ref[...],
                          preferred_element_type=jnp.float32)


def _scale_body(xw_ref, degp_ref, y_ref):
    y_ref[...] = xw_ref[...] * _dinv_from_parts(degp_ref[...])


def _lin2_body(y1_ref, accp_ref, degp_ref, w2_ref, b1_ref, y2_ref):
    dinv = _dinv_from_parts(degp_ref[...])
    acc = accp_ref[0] + accp_ref[1] + y1_ref[...]
    h1 = jax.nn.relu(dinv * acc + b1_ref[...])
    y2_ref[...] = jnp.dot(h1, w2_ref[...], preferred_element_type=jnp.float32) * dinv


def _make_final(n, n_pad):
    def final_body(y2_ref, accp_ref, degp_ref, b2_ref, batch_ref, wl_ref,
                   bl_ref, out_ref):
        dinv = _dinv_from_parts(degp_ref[...])
        acc = accp_ref[0] + accp_ref[1] + y2_ref[...]
        h2 = jax.nn.relu(dinv * acc + b2_ref[...])[:n]
        seg = lax.broadcasted_iota(jnp.int32, (1, G), 1)
        mask = (batch_ref[...] == seg).astype(jnp.float32)
        dn = (((0,), (0,)), ((), ()))
        sums = lax.dot_general(mask, h2, dn, preferred_element_type=jnp.float32)
        cnt = lax.dot_general(mask, jnp.ones((n, 1), jnp.float32), dn,
                              preferred_element_type=jnp.float32)
        pooled = sums / jnp.maximum(cnt, 1.0)
        out_ref[...] = (
            jnp.dot(pooled, wl_ref[...], preferred_element_type=jnp.float32)
            + bl_ref[...]
        )

    return final_body


def kernel(x, edge_index, batch, W1, b1, W2, b2, Wl, bl):
    n, d = x.shape
    e = edge_index.shape[1]

    n_pad = ((n + (8 * NS) - 1) // (8 * NS)) * (8 * NS)
    rpt = n_pad // NS
    cpp = (e + NS * CHUNK - 1) // (NS * CHUNK)
    n1 = ((cpp * 29) // 100 + 1) // 2 * 2
    n0 = (cpp - n1 + 1) // 2 * 2
    nck_scat = (n0, n1)
    e_pad = NS * CHUNK * max(cpp, n0 + n1) + CHUNK
    nck_deg = ((cpp + 1) // 2, cpp // 2)

    pad_e = e_pad - e
    src = jnp.concatenate(
        [edge_index[0], jnp.full((pad_e,), n, dtype=jnp.int32)])
    dst = jnp.concatenate(
        [edge_index[1], jnp.full((pad_e,), n, dtype=jnp.int32)])
    x_ext = jnp.concatenate(
        [x, jnp.zeros((n_pad - n, d), dtype=jnp.float32)], axis=0)
    ones_deg = jnp.ones((CHUNK, WDEG), dtype=jnp.float32)
    zeros_deg = jnp.zeros((rpt, WDEG), dtype=jnp.float32)
    zeros_row = jnp.zeros((rpt, d), dtype=jnp.float32)
    batch2 = batch.reshape(n, 1)
    b1r = b1.reshape(1, d)
    b2r = b2.reshape(1, d)
    blr = bl.reshape(1, 1)

    degp = _make_deg(n_pad, nck_deg, rpt)(dst, ones_deg, zeros_deg)

    xw1 = pl.pallas_call(
        _xw_body,
        out_shape=jax.ShapeDtypeStruct((n_pad, d), jnp.float32),
    )(x_ext, W1)
    y1 = pl.pallas_call(
        _scale_body,
        out_shape=jax.ShapeDtypeStruct((n_pad, d), jnp.float32),
    )(xw1, degp)

    scat = _make_scatter(n_pad, d, nck_scat, rpt)

    acc1 = scat(src, dst, y1, zeros_row)

    y2 = pl.pallas_call(
        _lin2_body,
        out_shape=jax.ShapeDtypeStruct((n_pad, d), jnp.float32),
    )(y1, acc1, degp, W2, b1r)

    acc2 = scat(src, dst, y2, zeros_row)

    out = pl.pallas_call(
        _make_final(n, n_pad),
        out_shape=jax.ShapeDtypeStruct((G, 1), jnp.float32),
    )(y2, acc2, degp, b2r, batch2, Wl, blr)

    return out.reshape(-1)
